# baseline (speedup 1.0000x reference)
import functools

import jax
import jax.numpy as jnp
from jax import lax
from jax.experimental import pallas as pl
from jax.experimental.pallas import tpu as pltpu
from jax.experimental.pallas import tpu_sc as plsc

_K = 16
_NW = 32
_CHUNK = 128


def _topk_into(pts, pselT, idx_ref, d_ref, k):
    n = pts.shape[1]
    m_rows = pselT.shape[0]
    inner = jnp.dot(pselT.astype(jnp.bfloat16), pts.astype(jnp.bfloat16),
                    preferred_element_type=jnp.float32)
    sqn = pts[0:1, :] * pts[0:1, :]
    sqm = pselT[:, 0:1] * pselT[:, 0:1]
    for c in range(1, 3):
        sqn = sqn + pts[c:c + 1, :] * pts[c:c + 1, :]
        sqm = sqm + pselT[:, c:c + 1] * pselT[:, c:c + 1]
    d_ref[...] = (sqm - 2.0 * inner) + sqn
    iota = lax.broadcasted_iota(jnp.int32, (m_rows, n), 1)
    for j in range(k):
        dcur = d_ref[...]
        dmin = jnp.min(dcur, axis=1, keepdims=True)
        am = jnp.min(jnp.where(dcur == dmin, iota, n), axis=1, keepdims=True)
        d_ref[...] = jnp.where(iota == am, jnp.inf, dcur)
        idx_ref[0, :, j:j + 1] = am


def _knn_s1_body(pts_ref, ps1_ref, idx1_ref, d1_ref, *, k):
    _topk_into(pts_ref[0], ps1_ref[0], idx1_ref, d1_ref, k)


def _knn_s1(pts, ps1):
    bsz = pts.shape[0]
    return pl.pallas_call(
        functools.partial(_knn_s1_body, k=_K),
        grid=(bsz,),
        in_specs=[
            pl.BlockSpec((1, 3, 2048), lambda b: (b, 0, 0)),
            pl.BlockSpec((1, 512, 3), lambda b: (b, 0, 0)),
        ],
        out_specs=pl.BlockSpec((1, 512, _K), lambda b: (b, 0, 0)),
        out_shape=jax.ShapeDtypeStruct((bsz, 512, _K), jnp.int32),
        scratch_shapes=[pltpu.VMEM((512, 2048), jnp.float32)],
    )(pts, ps1)


def _knn_rest_body(p1_ref, p2_ref, p3_ref, ps2_ref, ps3_ref,
                   idx2_ref, idx3_ref, idx4_ref, d2_ref, d3_ref, d4_ref, *, k):
    _topk_into(p1_ref[0], ps2_ref[0], idx2_ref, d2_ref, k)
    _topk_into(p2_ref[0], ps3_ref[0], idx3_ref, d3_ref, k)
    _topk_into(p3_ref[0], ps3_ref[0], idx4_ref, d4_ref, k)


def _knn_rest(p1, p2, p3, ps2, ps3):
    bsz = p1.shape[0]
    bat = lambda a: pl.BlockSpec((1,) + a.shape[1:],
                                 lambda b: (b,) + (0,) * (a.ndim - 1))
    args = (p1, p2, p3, ps2, ps3)
    mdims = (128, 32, 32)
    return pl.pallas_call(
        functools.partial(_knn_rest_body, k=_K),
        grid=(bsz,),
        in_specs=[bat(a) for a in args],
        out_specs=[pl.BlockSpec((1, m, _K), lambda b: (b, 0, 0)) for m in mdims],
        out_shape=[jax.ShapeDtypeStruct((bsz, m, _K), jnp.int32) for m in mdims],
        scratch_shapes=[
            pltpu.VMEM((128, 512), jnp.float32),
            pltpu.VMEM((32, 128), jnp.float32),
            pltpu.VMEM((32, 32), jnp.float32),
        ],
    )(*args)


def _sc_gather(table, idx_abs, c_out=None):
    tot = idx_abs.shape[0]
    c = table.shape[1]
    c_out = c if c_out is None else c_out
    rows_per_w = tot // _NW
    chunk = min(_CHUNK, rows_per_w, 32768 // c)
    n_chunks = rows_per_w // chunk
    assert rows_per_w % chunk == 0

    mesh = plsc.VectorSubcoreMesh(core_axis_name="c", subcore_axis_name="s")

    @functools.partial(
        pl.kernel, mesh=mesh,
        out_type=jax.ShapeDtypeStruct((tot, c_out), jnp.float32),
        scratch_types=[
            pltpu.VMEM((rows_per_w,), jnp.int32),
            pltpu.VMEM((2 * chunk, c), jnp.float32),
            pltpu.SemaphoreType.DMA,
        ],
    )
    def gather(table_hbm, idx_hbm, out_hbm, idx_v, rows_v, sem):
        wid = lax.axis_index("s") * 2 + lax.axis_index("c")
        base = wid * rows_per_w
        pltpu.sync_copy(idx_hbm.at[pl.ds(base, rows_per_w)], idx_v)

        def start(i):
            return pltpu.async_copy(
                table_hbm.at[idx_v.at[pl.ds(i * chunk, chunk)]],
                rows_v.at[pl.ds((i % 2) * chunk, chunk)], sem)

        def drain(i, cp):
            cp.wait()
            pltpu.sync_copy(
                rows_v.at[pl.ds((i % 2) * chunk, chunk), pl.ds(0, c_out)],
                out_hbm.at[pl.ds(base + i * chunk, chunk)])

        cps = [None] * n_chunks
        cps[0] = start(0)
        for i in range(1, n_chunks):
            cps[i] = start(i)
            drain(i - 1, cps[i - 1])
        drain(n_chunks - 1, cps[n_chunks - 1])

    return gather(table, idx_abs)


def _mlp_body(nb_ref, ftsel_ref, w1_ref, b1_ref, w2_ref, b2_ref, w3_ref, b3_ref,
              out_ref, *, k, reduce_n):
    bb = ftsel_ref.shape[0]
    m_rows, c_use = ftsel_ref.shape[1:]
    bdot = lambda x, w: jnp.dot(x.astype(jnp.bfloat16), w.astype(jnp.bfloat16),
                                preferred_element_type=jnp.float32)
    w1 = w1_ref[...]
    w2 = w2_ref[...]
    w3 = w3_ref[...]
    for bi in range(bb):
        ftsel = ftsel_ref[bi]
        nb = nb_ref[bi, :, :c_use]
        ctr = jnp.concatenate([ftsel] * k, axis=0)
        hcat = jnp.concatenate([ctr, nb - ctr], axis=1)
        h = jnp.maximum(bdot(hcat, w1) + b1_ref[...], 0.0)
        h = jnp.maximum(bdot(h, w2) + b2_ref[...], 0.0)
        h = jnp.maximum(bdot(h, w3) + b3_ref[...], 0.0)
        acc = h[:m_rows]
        for j in range(1, k):
            acc = jnp.maximum(acc, h[j * m_rows:(j + 1) * m_rows])
        if reduce_n:
            out_ref[bi] = jnp.max(acc, axis=0, keepdims=True)
        else:
            out_ref[bi] = acc


def _edge_mlp(nb, ftsel, layers, *, reduce_n=False, bb=8):
    (w1, b1), (w2, b2), (w3, b3) = layers
    bsz, m_rows, c = ftsel.shape
    c_nb = nb.shape[2]
    c3 = w3.shape[0]
    m_out = 1 if reduce_n else m_rows
    full = lambda a: pl.BlockSpec(a.shape, lambda b: (0,) * a.ndim)
    args = (nb, ftsel, w1.T, b1.reshape(1, -1), w2.T, b2.reshape(1, -1),
            w3.T, b3.reshape(1, -1))
    in_specs = [
        pl.BlockSpec((bb, _K * m_rows, c_nb), lambda b: (b, 0, 0)),
        pl.BlockSpec((bb, m_rows, c), lambda b: (b, 0, 0)),
    ] + [full(a) for a in args[2:]]
    return pl.pallas_call(
        functools.partial(_mlp_body, k=_K, reduce_n=reduce_n),
        grid=(bsz // bb,),
        in_specs=in_specs,
        out_specs=pl.BlockSpec((bb, m_out, c3), lambda b: (b, 0, 0)),
        out_shape=jax.ShapeDtypeStruct((bsz, m_out, c3), jnp.float32),
    )(*args)


def _abs_jmajor_idx(idx, n):
    bsz, m_rows, k = idx.shape
    idx_t = jnp.transpose(idx, (0, 2, 1)).reshape(bsz, k * m_rows)
    return (idx_t + (jnp.arange(bsz, dtype=jnp.int32) * n)[:, None]).reshape(-1)


def _pad_last(x, to):
    b, n, c = x.shape
    return jnp.concatenate([x, jnp.zeros((b, n, to - c), jnp.float32)], axis=2)


def _pad_w1(w1t):
    c1 = w1t.shape[1]
    z = jnp.zeros((13, c1), jnp.float32)
    return jnp.concatenate([w1t[:3], z, w1t[3:], z], axis=0)


def kernel(in_features, points, params):
    bsz = in_features.shape[0]
    ptsT = jnp.transpose(points, (0, 2, 1))
    pselT1 = ptsT[:, ::4]
    pselT2 = pselT1[:, ::4]
    pselT3 = pselT2[:, ::4]
    p1 = points[:, :, ::4]
    p2 = p1[:, :, ::4]
    p3 = p2[:, :, ::4]

    idx1 = _knn_s1(points, pselT1)

    (w1, b1), l2, l3 = params[0]
    ft0p = _pad_last(jnp.transpose(in_features, (0, 2, 1)), 16)
    table1 = _pad_last(ft0p, 128).reshape(bsz * 2048, 128)
    nb1 = _sc_gather(table1, _abs_jmajor_idx(idx1, 2048))

    idx2, idx3, idx4 = _knn_rest(p1, p2, p3, pselT2, pselT3)

    out1 = _edge_mlp(nb1.reshape(bsz, _K * 512, 128), ft0p[:, ::4],
                     ((_pad_w1(w1.T).T, b1), l2, l3), bb=2)

    nb2 = _sc_gather(out1.reshape(bsz * 512, 128), _abs_jmajor_idx(idx2, 512))
    out2 = _edge_mlp(nb2.reshape(bsz, _K * 128, 128), out1[:, ::4], params[1])

    nb3 = _sc_gather(out2.reshape(bsz * 128, 256), _abs_jmajor_idx(idx3, 128))
    out3 = _edge_mlp(nb3.reshape(bsz, _K * 32, 256), out2[:, ::4], params[2])

    nb4 = _sc_gather(out3.reshape(bsz * 32, 512), _abs_jmajor_idx(idx4, 32))
    out4 = _edge_mlp(nb4.reshape(bsz, _K * 32, 512), out3, params[3],
                     reduce_n=True)
    g = out4[:, 0, :]

    f1 = jnp.transpose(out1, (0, 2, 1))
    f2 = jnp.transpose(out2, (0, 2, 1))
    f3 = jnp.transpose(out3, (0, 2, 1))
    return ((f1, f2, f3), (p1, p2, p3), g)


# speedup vs baseline: 1.0822x; 1.0822x over previous
import functools

import jax
import jax.numpy as jnp
from jax import lax
from jax.experimental import pallas as pl
from jax.experimental.pallas import tpu as pltpu
from jax.experimental.pallas import tpu_sc as plsc

_K = 16
_NW = 32
_CHUNK = 128


def _topk_into(pts, pselT, idx_ref, d_ref, k):
    n = pts.shape[1]
    m_rows = pselT.shape[0]
    inner = jnp.dot(pselT.astype(jnp.bfloat16), pts.astype(jnp.bfloat16),
                    preferred_element_type=jnp.float32)
    sqn = pts[0:1, :] * pts[0:1, :]
    sqm = pselT[:, 0:1] * pselT[:, 0:1]
    for c in range(1, 3):
        sqn = sqn + pts[c:c + 1, :] * pts[c:c + 1, :]
        sqm = sqm + pselT[:, c:c + 1] * pselT[:, c:c + 1]
    d_ref[...] = (sqm - 2.0 * inner) + sqn
    iota = lax.broadcasted_iota(jnp.int32, (m_rows, n), 1)
    for j in range(k):
        dcur = d_ref[...]
        dmin = jnp.min(dcur, axis=1, keepdims=True)
        am = jnp.min(jnp.where(dcur == dmin, iota, n), axis=1, keepdims=True)
        d_ref[...] = jnp.where(iota == am, jnp.inf, dcur)
        idx_ref[0, :, j:j + 1] = am


def _knn_body(pts_ref, p1_ref, p2_ref, p3_ref, ps1_ref, ps2_ref, ps3_ref,
              idx1_ref, idx2_ref, idx3_ref, idx4_ref,
              d1_ref, d2_ref, d3_ref, d4_ref, *, k):
    _topk_into(pts_ref[0], ps1_ref[0], idx1_ref, d1_ref, k)
    _topk_into(p1_ref[0], ps2_ref[0], idx2_ref, d2_ref, k)
    _topk_into(p2_ref[0], ps3_ref[0], idx3_ref, d3_ref, k)
    _topk_into(p3_ref[0], ps3_ref[0], idx4_ref, d4_ref, k)


def _knn_all(pts, p1, p2, p3, ps1, ps2, ps3):
    bsz = pts.shape[0]
    bat = lambda a: pl.BlockSpec((1,) + a.shape[1:],
                                 lambda b: (b,) + (0,) * (a.ndim - 1))
    args = (pts, p1, p2, p3, ps1, ps2, ps3)
    mdims = (512, 128, 32, 32)
    return pl.pallas_call(
        functools.partial(_knn_body, k=_K),
        grid=(bsz,),
        in_specs=[bat(a) for a in args],
        out_specs=[pl.BlockSpec((1, m, _K), lambda b: (b, 0, 0)) for m in mdims],
        out_shape=[jax.ShapeDtypeStruct((bsz, m, _K), jnp.int32) for m in mdims],
        scratch_shapes=[
            pltpu.VMEM((512, 2048), jnp.float32),
            pltpu.VMEM((128, 512), jnp.float32),
            pltpu.VMEM((32, 128), jnp.float32),
            pltpu.VMEM((32, 32), jnp.float32),
        ],
    )(*args)


def _sc_gather(table, idx_abs, c_out=None):
    tot = idx_abs.shape[0]
    c = table.shape[1]
    c_out = c if c_out is None else c_out
    rows_per_w = tot // _NW
    chunk = min(_CHUNK, rows_per_w, 32768 // c)
    n_chunks = rows_per_w // chunk
    assert rows_per_w % chunk == 0

    mesh = plsc.VectorSubcoreMesh(core_axis_name="c", subcore_axis_name="s")

    @functools.partial(
        pl.kernel, mesh=mesh,
        out_type=jax.ShapeDtypeStruct((tot, c_out), jnp.float32),
        scratch_types=[
            pltpu.VMEM((rows_per_w,), jnp.int32),
            pltpu.VMEM((2 * chunk, c), jnp.float32),
            pltpu.SemaphoreType.DMA,
        ],
    )
    def gather(table_hbm, idx_hbm, out_hbm, idx_v, rows_v, sem):
        wid = lax.axis_index("s") * 2 + lax.axis_index("c")
        base = wid * rows_per_w
        pltpu.sync_copy(idx_hbm.at[pl.ds(base, rows_per_w)], idx_v)

        def start(i):
            return pltpu.async_copy(
                table_hbm.at[idx_v.at[pl.ds(i * chunk, chunk)]],
                rows_v.at[pl.ds((i % 2) * chunk, chunk)], sem)

        def drain(i, cp):
            cp.wait()
            pltpu.sync_copy(
                rows_v.at[pl.ds((i % 2) * chunk, chunk), pl.ds(0, c_out)],
                out_hbm.at[pl.ds(base + i * chunk, chunk)])

        cps = [None] * n_chunks
        cps[0] = start(0)
        for i in range(1, n_chunks):
            cps[i] = start(i)
            drain(i - 1, cps[i - 1])
        drain(n_chunks - 1, cps[n_chunks - 1])

    return gather(table, idx_abs)


def _mlp_body(nb_ref, ftsel_ref, w1_ref, b1_ref, w2_ref, b2_ref, w3_ref, b3_ref,
              out_ref, *, k, reduce_n):
    bb = ftsel_ref.shape[0]
    m_rows, c_use = ftsel_ref.shape[1:]
    bdot = lambda x, w: jnp.dot(x.astype(jnp.bfloat16), w.astype(jnp.bfloat16),
                                preferred_element_type=jnp.float32)
    w1 = w1_ref[...]
    w2 = w2_ref[...]
    w3 = w3_ref[...]
    for bi in range(bb):
        ftsel = ftsel_ref[bi]
        nb = nb_ref[bi, :, :c_use]
        ctr = jnp.concatenate([ftsel] * k, axis=0)
        hcat = jnp.concatenate([ctr, nb - ctr], axis=1)
        h = jnp.maximum(bdot(hcat, w1) + b1_ref[...], 0.0)
        h = jnp.maximum(bdot(h, w2) + b2_ref[...], 0.0)
        h = jnp.maximum(bdot(h, w3) + b3_ref[...], 0.0)
        acc = h[:m_rows]
        for j in range(1, k):
            acc = jnp.maximum(acc, h[j * m_rows:(j + 1) * m_rows])
        if reduce_n:
            out_ref[bi] = jnp.max(acc, axis=0, keepdims=True)
        else:
            out_ref[bi] = acc


def _edge_mlp(nb, ftsel, layers, *, reduce_n=False, bb=8):
    (w1, b1), (w2, b2), (w3, b3) = layers
    bsz, m_rows, c = ftsel.shape
    c_nb = nb.shape[2]
    c3 = w3.shape[0]
    m_out = 1 if reduce_n else m_rows
    full = lambda a: pl.BlockSpec(a.shape, lambda b: (0,) * a.ndim)
    args = (nb, ftsel, w1.T, b1.reshape(1, -1), w2.T, b2.reshape(1, -1),
            w3.T, b3.reshape(1, -1))
    in_specs = [
        pl.BlockSpec((bb, _K * m_rows, c_nb), lambda b: (b, 0, 0)),
        pl.BlockSpec((bb, m_rows, c), lambda b: (b, 0, 0)),
    ] + [full(a) for a in args[2:]]
    return pl.pallas_call(
        functools.partial(_mlp_body, k=_K, reduce_n=reduce_n),
        grid=(bsz // bb,),
        in_specs=in_specs,
        out_specs=pl.BlockSpec((bb, m_out, c3), lambda b: (b, 0, 0)),
        out_shape=jax.ShapeDtypeStruct((bsz, m_out, c3), jnp.float32),
    )(*args)


def _abs_jmajor_idx(idx, n):
    bsz, m_rows, k = idx.shape
    idx_t = jnp.transpose(idx, (0, 2, 1)).reshape(bsz, k * m_rows)
    return (idx_t + (jnp.arange(bsz, dtype=jnp.int32) * n)[:, None]).reshape(-1)


def _pad_last(x, to):
    b, n, c = x.shape
    return jnp.concatenate([x, jnp.zeros((b, n, to - c), jnp.float32)], axis=2)


def _pad_w1(w1t):
    c1 = w1t.shape[1]
    z = jnp.zeros((13, c1), jnp.float32)
    return jnp.concatenate([w1t[:3], z, w1t[3:], z], axis=0)


def kernel(in_features, points, params):
    bsz = in_features.shape[0]
    ptsT = jnp.transpose(points, (0, 2, 1))
    pselT1 = ptsT[:, ::4]
    pselT2 = pselT1[:, ::4]
    pselT3 = pselT2[:, ::4]
    p1 = points[:, :, ::4]
    p2 = p1[:, :, ::4]
    p3 = p2[:, :, ::4]

    idx1, idx2, idx3, idx4 = _knn_all(points, p1, p2, p3,
                                      pselT1, pselT2, pselT3)

    (w1, b1), l2, l3 = params[0]
    ft0p = _pad_last(jnp.transpose(in_features, (0, 2, 1)), 16)
    table1 = _pad_last(ft0p, 128).reshape(bsz * 2048, 128)
    nb1 = _sc_gather(table1, _abs_jmajor_idx(idx1, 2048))
    out1 = _edge_mlp(nb1.reshape(bsz, _K * 512, 128), ft0p[:, ::4],
                     ((_pad_w1(w1.T).T, b1), l2, l3), bb=2)

    nb2 = _sc_gather(out1.reshape(bsz * 512, 128), _abs_jmajor_idx(idx2, 512))
    out2 = _edge_mlp(nb2.reshape(bsz, _K * 128, 128), out1[:, ::4], params[1])

    nb3 = _sc_gather(out2.reshape(bsz * 128, 256), _abs_jmajor_idx(idx3, 128))
    out3 = _edge_mlp(nb3.reshape(bsz, _K * 32, 256), out2[:, ::4], params[2])

    nb4 = _sc_gather(out3.reshape(bsz * 32, 512), _abs_jmajor_idx(idx4, 32))
    out4 = _edge_mlp(nb4.reshape(bsz, _K * 32, 512), out3, params[3],
                     reduce_n=True)
    g = out4[:, 0, :]

    f1 = jnp.transpose(out1, (0, 2, 1))
    f2 = jnp.transpose(out2, (0, 2, 1))
    f3 = jnp.transpose(out3, (0, 2, 1))
    return ((f1, f2, f3), (p1, p2, p3), g)
